# Initial kernel scaffold; baseline (speedup 1.0000x reference)
import jax
import jax.numpy as jnp
from jax.experimental import pallas as pl


def kernel(update_values, update_indices, update_weights, volume, volume_weights):
    raise NotImplementedError("write your pallas kernel here")


# speedup vs baseline: 1.1887x; 1.1887x over previous
import jax
import jax.numpy as jnp
from jax.experimental import pallas as pl

INIT_VALUE = 0.04
M = 256 ** 3


def _merge_body(vol_ref, vw_ref, vacc_ref, wacc_ref, nv_ref, nw_ref):
    vol = vol_ref[...]
    vw = vw_ref[...]
    vacc = vacc_ref[...]
    wacc = wacc_ref[...]
    new_w = vw + wacc
    denom = jnp.maximum(new_w, 1e-8)
    fused = (vol * vw + vacc) / denom
    nv = jnp.where(wacc > 0.0, fused, vol)
    nv_ref[...] = jnp.clip(nv, -INIT_VALUE, INIT_VALUE)
    nw_ref[...] = jnp.minimum(new_w, 255.0)


def kernel(update_values, update_indices, update_weights, volume, volume_weights):
    uv = jnp.clip(update_values, -INIT_VALUE, INIT_VALUE)
    ext = (uv[..., None] * update_weights).reshape(-1)
    idx = update_indices.reshape(-1)
    wfl = update_weights.reshape(-1)
    vacc = jnp.zeros_like(volume).at[idx].add(ext)
    wacc = jnp.zeros_like(volume_weights).at[idx].add(wfl)

    R, C = 131072, 128
    BR = 1024
    vol2 = volume.reshape(R, C)
    vw2 = volume_weights.reshape(R, C)
    vacc2 = vacc.reshape(R, C)
    wacc2 = wacc.reshape(R, C)
    spec = pl.BlockSpec((BR, C), lambda i: (i, 0))
    nv, nw = pl.pallas_call(
        _merge_body,
        grid=(R // BR,),
        in_specs=[spec, spec, spec, spec],
        out_specs=[spec, spec],
        out_shape=[
            jax.ShapeDtypeStruct((R, C), jnp.float32),
            jax.ShapeDtypeStruct((R, C), jnp.float32),
        ],
    )(vol2, vw2, vacc2, wacc2)
    return nv.reshape(-1), nw.reshape(-1)


# speedup vs baseline: 1.7821x; 1.4991x over previous
import jax
import jax.numpy as jnp
from jax import lax
from jax.experimental import pallas as pl
from jax.experimental.pallas import tpu as pltpu
from jax.experimental.pallas import tpu_sc as plsc

INIT_VALUE = 0.04
M = 256 ** 3
NRAYS, T, K = 76800, 9, 8
U = NRAYS * T * K
NC, NS = 2, 16
NW = NC * NS
RPT = U // NW
UPT = RPT // K
BIN_BITS = 19
BINSZ = 1 << BIN_BITS
NBINS = M >> BIN_BITS
BPSC = NBINS // NC
WIN_A = 3200
NWIN_A = RPT // WIN_A
NCH_A = WIN_A // 128
CHB = 1024
NCHB = CHB // 128
PAD = CHB + 8
WM = 4096
VPT = BINSZ // NS
NWM = VPT // WM
ZBUF = 8192

_mesh = plsc.VectorSubcoreMesh(core_axis_name="c", subcore_axis_name="s")


def _partition_body(uv_hbm, idx_hbm, w_hbm,
                    bloc_hbm, bv_hbm, bw_hbm, tbl_hbm,
                    idx_win, w_win, uv_win, loc_win, v_win, dest2d,
                    hist, cursor, tblrow, sem):
    cid = lax.axis_index("c")
    sid = lax.axis_index("s")
    wid = sid * NC + cid
    lane = lax.iota(jnp.int32, 16)
    ones = jnp.ones((16,), jnp.int32)
    rbase = wid * RPT

    zz = jnp.zeros((16,), jnp.int32)
    for j in range(NBINS * 16 // 16):
        hist[pl.ds(j * 16, 16)] = zz

    def w1(win, _):
        base = pl.multiple_of(rbase + win * WIN_A, 8)
        pltpu.sync_copy(idx_hbm.at[pl.ds(base, WIN_A)], idx_win)
        for c in range(NCH_A):
            def h1(q, _):
                iv = idx_win[pl.ds(c * 128 + q * 16, 16)]
                hi = lax.shift_right_logical(iv, BIN_BITS - 4)
                addr = jnp.bitwise_or(jnp.bitwise_and(hi, (NBINS - 1) * 16),
                                      lane)
                plsc.addupdate_scatter(hist, [addr], ones)
                return 0
            lax.fori_loop(0, 8, h1, 0)
        return 0
    lax.fori_loop(0, NWIN_A, w1, 0)

    def pf(j, carry):
        v = hist[pl.ds(j * 16, 16)]
        c = plsc.cumsum(v)
        cursor[pl.ds(j * 16, 16)] = c - v + carry
        return carry + jnp.sum(v)
    lax.fori_loop(0, NBINS * 16 // 16, pf, jnp.int32(0))

    tblrow[pl.ds(0, 16)] = plsc.load_gather(cursor, [lane * 16])
    tblrow[pl.ds(16, 16)] = plsc.load_gather(cursor, [lane * 16 + 256])
    pltpu.sync_copy(tblrow, tbl_hbm.at[pl.ds(pl.multiple_of(wid * 32, 8), 32)])

    def w2(win, _):
        base = pl.multiple_of(rbase + win * WIN_A, 8)
        ubase = pl.multiple_of(wid * UPT + win * (WIN_A // 8), 8)
        pltpu.sync_copy(idx_hbm.at[pl.ds(base, WIN_A)], idx_win)
        pltpu.sync_copy(w_hbm.at[pl.ds(base, WIN_A)], w_win)
        pltpu.sync_copy(uv_hbm.at[pl.ds(ubase, WIN_A // 8)], uv_win)
        lane8 = lax.shift_right_logical(lane, 3)
        for c in range(NCH_A):
            def c2(q, _):
                off = c * 128 + q * 16
                iv = idx_win[pl.ds(off, 16)]
                wv = w_win[pl.ds(off, 16)]
                uvv = plsc.load_gather(uv_win, [c * 16 + 2 * q + lane8])
                uvv = jnp.clip(uvv, -INIT_VALUE, INIT_VALUE)
                hi = lax.shift_right_logical(iv, BIN_BITS - 4)
                addr = jnp.bitwise_or(jnp.bitwise_and(hi, (NBINS - 1) * 16),
                                      lane)
                pos = plsc.load_gather(cursor, [addr])
                plsc.store_scatter(cursor, [addr], pos + 1)
                loc_win[pl.ds(off, 16)] = jnp.bitwise_and(iv, BINSZ - 1)
                v_win[pl.ds(off, 16)] = uvv * wv
                dest2d[c, pl.ds(q * 16, 16)] = pos + rbase
                return 0
            lax.fori_loop(0, 8, c2, 0)
        descs = []
        for c in range(NCH_A):
            di = plsc.Indices(dest2d.at[c])
            s = pl.ds(c * 128, 128)
            descs.append(pltpu.async_copy(loc_win.at[s], bloc_hbm.at[di], sem))
            descs.append(pltpu.async_copy(v_win.at[s], bv_hbm.at[di], sem))
            descs.append(pltpu.async_copy(w_win.at[s], bw_hbm.at[di], sem))
        for d in descs:
            d.wait()
        return 0
    lax.fori_loop(0, NWIN_A, w2, 0)


def _accmerge_body(bloc_hbm, bv_hbm, bw_hbm, tbl_hbm, vol_hbm, vw_hbm,
                   nv_hbm, nw_hbm,
                   vacc_sh, wacc_sh, tbl_sh, tbl_smem,
                   lidx_win, lv_win, lw_win, ilist2d,
                   vacc_win, wacc_win, vol_win, vww_win, nv_win, nww_win,
                   zbuf, sem_g, sem_s, sem_o):
    cid = lax.axis_index("c")
    sid = lax.axis_index("s")
    lane = lax.iota(jnp.int32, 16)

    @pl.when(sid == 0)
    def _copy_tbl():
        pltpu.sync_copy(tbl_hbm, tbl_sh)
    plsc.subcore_barrier()
    pltpu.sync_copy(tbl_sh, tbl_smem)

    zf = jnp.zeros((16,), jnp.float32)
    def zb(j, _):
        zbuf[pl.ds(j * 16, 16)] = zf
        return 0
    lax.fori_loop(0, ZBUF // 16, zb, 0)

    def per_bin(k, _):
        b = k * NC + cid

        def z1(j, _):
            off = pl.multiple_of(sid * VPT + j * ZBUF, 8)
            pltpu.sync_copy(zbuf, vacc_sh.at[pl.ds(off, ZBUF)])
            pltpu.sync_copy(zbuf, wacc_sh.at[pl.ds(off, ZBUF)])
            return 0
        lax.fori_loop(0, VPT // ZBUF, z1, 0)
        plsc.subcore_barrier()

        def src_tile(t):
            start = tbl_smem[t * NBINS + b]
            nxt = jnp.where(b == NBINS - 1, 0, t * NBINS + b + 1)
            end = jnp.where(b == NBINS - 1, RPT, tbl_smem[nxt])
            gstart = t * RPT + start
            gend = t * RPT + end
            astart = jnp.bitwise_and(gstart, -8)
            nch = lax.shift_right_logical(gend - astart + CHB - 1, 10)

            def chunk(kk, _):
                cbase = pl.multiple_of(astart + kk * CHB, 8)
                d1 = pltpu.async_copy(bloc_hbm.at[pl.ds(cbase, CHB)],
                                      lidx_win, sem_g)
                d2 = pltpu.async_copy(bv_hbm.at[pl.ds(cbase, CHB)],
                                      lv_win, sem_g)
                d3 = pltpu.async_copy(bw_hbm.at[pl.ds(cbase, CHB)],
                                      lw_win, sem_g)
                d1.wait(); d2.wait(); d3.wait()
                dump = BINSZ + lane
                for c in range(NCHB):
                    for q in range(8):
                        off = c * 128 + q * 16
                        posv = cbase + off + lane
                        okm = jnp.logical_and(posv >= gstart, posv < gend)
                        lv = lidx_win[pl.ds(off, 16)]
                        ilist2d[c, pl.ds(q * 16, 16)] = jnp.where(okm, lv,
                                                                  dump)
                for c in range(NCHB):
                    di = plsc.Indices(ilist2d.at[c])
                    s = pl.ds(c * 128, 128)
                    e1 = pltpu.async_copy(lv_win.at[s], vacc_sh.at[di],
                                          sem_s, add=True)
                    e2 = pltpu.async_copy(lw_win.at[s], wacc_sh.at[di],
                                          sem_s, add=True)
                    e1.wait()
                    e2.wait()
                return 0
            lax.fori_loop(0, nch, chunk, 0)
        src_tile(sid)
        src_tile(sid + NS)
        plsc.subcore_barrier()

        def mw(w, _):
            sb = pl.multiple_of(sid * VPT + w * WM, 8)
            gb = pl.multiple_of(b * BINSZ + sid * VPT + w * WM, 8)
            pltpu.sync_copy(vacc_sh.at[pl.ds(sb, WM)], vacc_win)
            pltpu.sync_copy(wacc_sh.at[pl.ds(sb, WM)], wacc_win)
            pltpu.sync_copy(vol_hbm.at[pl.ds(gb, WM)], vol_win)
            pltpu.sync_copy(vw_hbm.at[pl.ds(gb, WM)], vww_win)

            def mc(c, _):
                for q in range(8):
                    off = c * 128 + q * 16
                    vol = vol_win[pl.ds(off, 16)]
                    vw = vww_win[pl.ds(off, 16)]
                    va = vacc_win[pl.ds(off, 16)]
                    wa = wacc_win[pl.ds(off, 16)]
                    nwv = vw + wa
                    den = jnp.maximum(nwv, 1e-8)
                    fused = (vol * vw + va) / den
                    nv = jnp.where(wa > 0.0, fused, vol)
                    nv_win[pl.ds(off, 16)] = jnp.clip(nv, -INIT_VALUE,
                                                      INIT_VALUE)
                    nww_win[pl.ds(off, 16)] = jnp.minimum(nwv, 255.0)
                return 0
            lax.fori_loop(0, WM // 128, mc, 0)
            pltpu.sync_copy(nv_win, nv_hbm.at[pl.ds(gb, WM)])
            pltpu.sync_copy(nww_win, nw_hbm.at[pl.ds(gb, WM)])
            return 0
        lax.fori_loop(0, NWM, mw, 0)
        plsc.subcore_barrier()
        return 0
    lax.fori_loop(0, BPSC, per_bin, 0)


_partition = pl.kernel(
    _partition_body,
    out_type=[
        jax.ShapeDtypeStruct((U + PAD,), jnp.int32),
        jax.ShapeDtypeStruct((U + PAD,), jnp.float32),
        jax.ShapeDtypeStruct((U + PAD,), jnp.float32),
        jax.ShapeDtypeStruct((NW * NBINS,), jnp.int32),
    ],
    mesh=_mesh,
    compiler_params=pltpu.CompilerParams(needs_layout_passes=False),
    scratch_types=[
        pltpu.VMEM((WIN_A,), jnp.int32),
        pltpu.VMEM((WIN_A,), jnp.float32),
        pltpu.VMEM((WIN_A // 8,), jnp.float32),
        pltpu.VMEM((WIN_A,), jnp.int32),
        pltpu.VMEM((WIN_A,), jnp.float32),
        pltpu.VMEM((NCH_A, 128), jnp.int32),
        pltpu.VMEM((NBINS * 16,), jnp.int32),
        pltpu.VMEM((NBINS * 16,), jnp.int32),
        pltpu.VMEM((32,), jnp.int32),
        pltpu.SemaphoreType.DMA,
    ],
)

_accmerge = pl.kernel(
    _accmerge_body,
    out_type=[
        jax.ShapeDtypeStruct((M,), jnp.float32),
        jax.ShapeDtypeStruct((M,), jnp.float32),
    ],
    mesh=_mesh,
    compiler_params=pltpu.CompilerParams(needs_layout_passes=False),
    scratch_types=[
        pltpu.VMEM_SHARED((BINSZ + 16,), jnp.float32),
        pltpu.VMEM_SHARED((BINSZ + 16,), jnp.float32),
        pltpu.VMEM_SHARED((NW * NBINS,), jnp.int32),
        pltpu.SMEM((NW * NBINS,), jnp.int32),
        pltpu.VMEM((CHB,), jnp.int32),
        pltpu.VMEM((CHB,), jnp.float32),
        pltpu.VMEM((CHB,), jnp.float32),
        pltpu.VMEM((NCHB, 128), jnp.int32),
        pltpu.VMEM((WM,), jnp.float32),
        pltpu.VMEM((WM,), jnp.float32),
        pltpu.VMEM((WM,), jnp.float32),
        pltpu.VMEM((WM,), jnp.float32),
        pltpu.VMEM((WM,), jnp.float32),
        pltpu.VMEM((WM,), jnp.float32),
        pltpu.VMEM((ZBUF,), jnp.float32),
        pltpu.SemaphoreType.DMA,
        pltpu.SemaphoreType.DMA,
        pltpu.SemaphoreType.DMA,
    ],
)


def kernel(update_values, update_indices, update_weights, volume,
           volume_weights):
    uvf = update_values.reshape(-1)
    idxf = update_indices.reshape(-1)
    wf = update_weights.reshape(-1)
    b_loc, b_v, b_w, tbl = _partition(uvf, idxf, wf)
    nv, nw = _accmerge(b_loc, b_v, b_w, tbl, volume, volume_weights)
    return nv, nw
